# baseline (speedup 1.0000x reference)
import jax
import jax.numpy as jnp
from jax.experimental import pallas as pl
from jax.experimental.pallas import tpu as pltpu

N_EXPERTS = 64
K_ACTIVE = 8
BT = 1024
NBUF = 3


def _router_step(x_ref, topi_ref, w_out_ref, w_vmem):
    logits_t = jax.lax.dot_general(
        w_vmem[...], x_ref[...],
        dimension_numbers=(((1,), (1,)), ((), ())),
        preferred_element_type=jnp.float32,
    )

    b = jax.lax.bitcast_convert_type(logits_t, jnp.int32)
    exp_i = jax.lax.broadcasted_iota(jnp.int32, logits_t.shape, 0)
    exp_code = jnp.where(b >= 0, (N_EXPERTS - 1) - exp_i, exp_i)
    key = jax.lax.bitcast_convert_type(
        jnp.bitwise_or(jnp.bitwise_and(b, -N_EXPERTS), exp_code),
        jnp.float32)

    neg_inf = jnp.float32(-jnp.inf)
    kms = []
    for j in range(K_ACTIVE):
        km = jnp.max(key, axis=0, keepdims=True)
        kms.append(km)
        if j + 1 < K_ACTIVE:
            key = jnp.where(key == km, neg_inf, key)

    kk = jnp.concatenate(kms, axis=0)
    kb = jax.lax.bitcast_convert_type(kk, jnp.int32)
    id6 = jnp.bitwise_and(kb, N_EXPERTS - 1)
    topi_t = jnp.where(kb < 0, id6, (N_EXPERTS - 1) - id6)
    topv_t = jax.lax.bitcast_convert_type(
        jnp.bitwise_and(kb, -N_EXPERTS), jnp.float32)

    e = jnp.exp(topv_t - topv_t[:1, :])
    w_t = e / jnp.sum(e, axis=0, keepdims=True)

    topi_ref[...] = topi_t.T
    w_out_ref[...] = w_t.T


def _router_body(x_hbm, w_hbm, topi_hbm, wout_hbm, w_vmem, sem):
    cp = pltpu.make_async_copy(w_hbm, w_vmem, sem)
    cp.start()
    cp.wait()

    n_tokens, d_model = x_hbm.shape

    def inner(x_ref, topi_ref, wout_ref):
        _router_step(x_ref, topi_ref, wout_ref, w_vmem)

    pltpu.emit_pipeline(
        inner,
        grid=(n_tokens // BT,),
        in_specs=[
            pl.BlockSpec((BT, d_model), lambda i: (i, 0),
                         pipeline_mode=pl.Buffered(buffer_count=NBUF,
                                                   use_lookahead=True)),
        ],
        out_specs=[
            pl.BlockSpec((BT, K_ACTIVE), lambda i: (i, 0)),
            pl.BlockSpec((BT, K_ACTIVE), lambda i: (i, 0)),
        ],
    )(x_hbm, topi_hbm, wout_hbm)


@jax.jit
def kernel(x, W):
    n_tokens, d_model = x.shape
    topi, w = pl.pallas_call(
        _router_body,
        in_specs=[
            pl.BlockSpec(memory_space=pl.ANY),
            pl.BlockSpec(memory_space=pl.ANY),
        ],
        out_specs=[
            pl.BlockSpec(memory_space=pl.ANY),
            pl.BlockSpec(memory_space=pl.ANY),
        ],
        out_shape=[
            jax.ShapeDtypeStruct((n_tokens, K_ACTIVE), jnp.int32),
            jax.ShapeDtypeStruct((n_tokens, K_ACTIVE), jnp.float32),
        ],
        scratch_shapes=[
            pltpu.VMEM((N_EXPERTS, d_model), jnp.float32),
            pltpu.SemaphoreType.DMA,
        ],
    )(x, W)
    return topi, w


# speedup vs baseline: 1.0042x; 1.0042x over previous
import jax
import jax.numpy as jnp
from jax.experimental import pallas as pl
from jax.experimental.pallas import tpu as pltpu

N_EXPERTS = 64
K_ACTIVE = 8
BT = 512
NBUF = 5


def _router_step(x_ref, topi_ref, w_out_ref, w_vmem):
    logits_t = jax.lax.dot_general(
        w_vmem[...], x_ref[...],
        dimension_numbers=(((1,), (1,)), ((), ())),
        preferred_element_type=jnp.float32,
    )

    b = jax.lax.bitcast_convert_type(logits_t, jnp.int32)
    exp_i = jax.lax.broadcasted_iota(jnp.int32, logits_t.shape, 0)
    exp_code = jnp.where(b >= 0, (N_EXPERTS - 1) - exp_i, exp_i)
    key = jax.lax.bitcast_convert_type(
        jnp.bitwise_or(jnp.bitwise_and(b, -N_EXPERTS), exp_code),
        jnp.float32)

    neg_inf = jnp.float32(-jnp.inf)
    kms = []
    for j in range(K_ACTIVE):
        km = jnp.max(key, axis=0, keepdims=True)
        kms.append(km)
        if j + 1 < K_ACTIVE:
            key = jnp.where(key == km, neg_inf, key)

    kk = jnp.concatenate(kms, axis=0)
    kb = jax.lax.bitcast_convert_type(kk, jnp.int32)
    id6 = jnp.bitwise_and(kb, N_EXPERTS - 1)
    topi_t = jnp.where(kb < 0, id6, (N_EXPERTS - 1) - id6)
    topv_t = jax.lax.bitcast_convert_type(
        jnp.bitwise_and(kb, -N_EXPERTS), jnp.float32)

    e = jnp.exp(topv_t - topv_t[:1, :])
    w_t = e / jnp.sum(e, axis=0, keepdims=True)

    topi_ref[...] = topi_t.T
    w_out_ref[...] = w_t.T


def _router_body(x_hbm, w_hbm, topi_hbm, wout_hbm, w_vmem, sem):
    cp = pltpu.make_async_copy(w_hbm, w_vmem, sem)
    cp.start()
    cp.wait()

    n_tokens, d_model = x_hbm.shape

    def inner(x_ref, topi_ref, wout_ref):
        _router_step(x_ref, topi_ref, wout_ref, w_vmem)

    pltpu.emit_pipeline(
        inner,
        grid=(n_tokens // BT,),
        in_specs=[
            pl.BlockSpec((BT, d_model), lambda i: (i, 0),
                         pipeline_mode=pl.Buffered(buffer_count=NBUF,
                                                   use_lookahead=True)),
        ],
        out_specs=[
            pl.BlockSpec((BT, K_ACTIVE), lambda i: (i, 0)),
            pl.BlockSpec((BT, K_ACTIVE), lambda i: (i, 0)),
        ],
    )(x_hbm, topi_hbm, wout_hbm)


@jax.jit
def kernel(x, W):
    n_tokens, d_model = x.shape
    topi, w = pl.pallas_call(
        _router_body,
        in_specs=[
            pl.BlockSpec(memory_space=pl.ANY),
            pl.BlockSpec(memory_space=pl.ANY),
        ],
        out_specs=[
            pl.BlockSpec(memory_space=pl.ANY),
            pl.BlockSpec(memory_space=pl.ANY),
        ],
        out_shape=[
            jax.ShapeDtypeStruct((n_tokens, K_ACTIVE), jnp.int32),
            jax.ShapeDtypeStruct((n_tokens, K_ACTIVE), jnp.float32),
        ],
        scratch_shapes=[
            pltpu.VMEM((N_EXPERTS, d_model), jnp.float32),
            pltpu.SemaphoreType.DMA,
        ],
    )(x, W)
    return topi, w


# speedup vs baseline: 1.0104x; 1.0061x over previous
import jax
import jax.numpy as jnp
from jax.experimental import pallas as pl
from jax.experimental.pallas import tpu as pltpu

N_EXPERTS = 64
K_ACTIVE = 8
BT = 256
NBUF = 10


def _router_step(x_ref, topi_ref, w_out_ref, w_vmem):
    logits_t = jax.lax.dot_general(
        w_vmem[...], x_ref[...],
        dimension_numbers=(((1,), (1,)), ((), ())),
        preferred_element_type=jnp.float32,
    )

    b = jax.lax.bitcast_convert_type(logits_t, jnp.int32)
    exp_i = jax.lax.broadcasted_iota(jnp.int32, logits_t.shape, 0)
    exp_code = jnp.where(b >= 0, (N_EXPERTS - 1) - exp_i, exp_i)
    key = jax.lax.bitcast_convert_type(
        jnp.bitwise_or(jnp.bitwise_and(b, -N_EXPERTS), exp_code),
        jnp.float32)

    neg_inf = jnp.float32(-jnp.inf)
    kms = []
    for j in range(K_ACTIVE):
        km = jnp.max(key, axis=0, keepdims=True)
        kms.append(km)
        if j + 1 < K_ACTIVE:
            key = jnp.where(key == km, neg_inf, key)

    kk = jnp.concatenate(kms, axis=0)
    kb = jax.lax.bitcast_convert_type(kk, jnp.int32)
    id6 = jnp.bitwise_and(kb, N_EXPERTS - 1)
    topi_t = jnp.where(kb < 0, id6, (N_EXPERTS - 1) - id6)
    topv_t = jax.lax.bitcast_convert_type(
        jnp.bitwise_and(kb, -N_EXPERTS), jnp.float32)

    e = jnp.exp(topv_t - topv_t[:1, :])
    w_t = e / jnp.sum(e, axis=0, keepdims=True)

    topi_ref[...] = topi_t.T
    w_out_ref[...] = w_t.T


def _router_body(x_hbm, w_hbm, topi_hbm, wout_hbm, w_vmem, sem):
    cp = pltpu.make_async_copy(w_hbm, w_vmem, sem)
    cp.start()
    cp.wait()

    n_tokens, d_model = x_hbm.shape

    def inner(x_ref, topi_ref, wout_ref):
        _router_step(x_ref, topi_ref, wout_ref, w_vmem)

    pltpu.emit_pipeline(
        inner,
        grid=(n_tokens // BT,),
        in_specs=[
            pl.BlockSpec((BT, d_model), lambda i: (i, 0),
                         pipeline_mode=pl.Buffered(buffer_count=NBUF,
                                                   use_lookahead=True)),
        ],
        out_specs=[
            pl.BlockSpec((BT, K_ACTIVE), lambda i: (i, 0)),
            pl.BlockSpec((BT, K_ACTIVE), lambda i: (i, 0)),
        ],
    )(x_hbm, topi_hbm, wout_hbm)


@jax.jit
def kernel(x, W):
    n_tokens, d_model = x.shape
    topi, w = pl.pallas_call(
        _router_body,
        in_specs=[
            pl.BlockSpec(memory_space=pl.ANY),
            pl.BlockSpec(memory_space=pl.ANY),
        ],
        out_specs=[
            pl.BlockSpec(memory_space=pl.ANY),
            pl.BlockSpec(memory_space=pl.ANY),
        ],
        out_shape=[
            jax.ShapeDtypeStruct((n_tokens, K_ACTIVE), jnp.int32),
            jax.ShapeDtypeStruct((n_tokens, K_ACTIVE), jnp.float32),
        ],
        scratch_shapes=[
            pltpu.VMEM((N_EXPERTS, d_model), jnp.float32),
            pltpu.SemaphoreType.DMA,
        ],
    )(x, W)
    return topi, w
